# baseline (speedup 1.0000x reference)
import functools
import math

import jax
import jax.numpy as jnp
from jax import lax
from jax.experimental import pallas as pl
from jax.experimental.pallas import tpu as pltpu
from jax.experimental.pallas import tpu_sc as plsc

_B = 8
_NN = 50
_NNPAD = 64
_BIG = 3.0e38


_MAXI = 2147483647


def _knn_body(xall_ref, xblk_ref, out_ref, k_ref, *, rows, v, cw):
    xb = xblk_ref[0]
    qb = jnp.sum(xb * xb, axis=1)[:, None]
    rowg = pl.program_id(1) * rows + lax.broadcasted_iota(
        jnp.int32, (rows, cw), 0)
    nchunks = v // cw

    for c in range(nchunks):
        xa = xall_ref[0, pl.ds(c * cw, cw), :]
        inner = lax.dot_general(xb, xa, (((1,), (1,)), ((), ())),
                                preferred_element_type=jnp.float32)
        qa = jnp.sum(xa * xa, axis=1)[None, :]
        dist = jnp.maximum(qa + qb - 2.0 * inner, 0.0)
        bits = lax.bitcast_convert_type(dist, jnp.int32)
        colc = c * cw + lax.broadcasted_iota(jnp.int32, (rows, cw), 1)
        key = ((bits + 1024) & ~2047) | colc
        k_ref[:, pl.ds(c * cw, cw)] = jnp.where(colc == rowg, _MAXI, key)

    col64 = lax.broadcasted_iota(jnp.int32, (rows, _NNPAD), 1)

    def step(k, carry):
        res, m = carry
        acc = jnp.full((rows, 128), _MAXI, jnp.int32)
        for c in range(nchunks):
            kch = k_ref[:, pl.ds(c * cw, cw)]
            kch = jnp.where(kch == m, _MAXI, kch)
            k_ref[:, pl.ds(c * cw, cw)] = kch
            for s in range(cw // 128):
                acc = jnp.minimum(acc, kch[:, s * 128:(s + 1) * 128])
        m = jnp.min(acc, axis=1, keepdims=True)
        return jnp.where(col64 == k, m & 2047, res), m

    res0 = jnp.zeros((rows, _NNPAD), jnp.int32)
    m0 = jnp.full((rows, 1), _MAXI, jnp.int32)
    (res, _) = lax.fori_loop(0, _NN, step, (res0, m0), unroll=1)
    out_ref[0] = res


def _knn(verts):
    b, v, _ = verts.shape
    rows = 64 if v > 512 else min(v, 128)
    cw = min(v, 512)
    vp = jnp.pad(verts, ((0, 0), (0, 0), (0, 125)))
    return pl.pallas_call(
        functools.partial(_knn_body, rows=rows, v=v, cw=cw),
        grid=(b, v // rows),
        in_specs=[
            pl.BlockSpec((1, v, 128), lambda i, j: (i, 0, 0)),
            pl.BlockSpec((1, rows, 128), lambda i, j: (i, j, 0)),
        ],
        out_specs=pl.BlockSpec((1, rows, _NNPAD), lambda i, j: (i, j, 0)),
        out_shape=jax.ShapeDtypeStruct((b, v, _NNPAD), jnp.int32),
        scratch_shapes=[pltpu.VMEM((rows, v), jnp.int32)],
    )(vp, vp)


def _mm_body(x_ref, w_ref, b_ref, o_ref):
    o_ref[...] = jnp.dot(x_ref[...], w_ref[...],
                         preferred_element_type=jnp.float32) + b_ref[...]


def _mm(x, w, b):
    m, kdim = x.shape
    n = w.shape[1]
    mblk = min(m, 2048)
    return pl.pallas_call(
        _mm_body,
        grid=(m // mblk,),
        in_specs=[
            pl.BlockSpec((mblk, kdim), lambda i: (i, 0)),
            pl.BlockSpec((kdim, n), lambda i: (0, 0)),
            pl.BlockSpec((1, n), lambda i: (0, 0)),
        ],
        out_specs=pl.BlockSpec((mblk, n), lambda i: (i, 0)),
        out_shape=jax.ShapeDtypeStruct((m, n), jnp.float32),
    )(x, w, b[None, :])


def _final_body(x_ref, w_ref, b_ref, o_ref):
    rows = []
    for i in range(_B):
        rows.append(jnp.max(x_ref[i], axis=0)[None, :])
    gmax = jnp.concatenate(rows, axis=0)
    o_ref[...] = jnp.dot(gmax, w_ref[...],
                         preferred_element_type=jnp.float32) + b_ref[...]


def _final(fm4, w_fc, b_fc):
    return pl.pallas_call(
        _final_body,
        out_shape=jax.ShapeDtypeStruct((_B, w_fc.shape[1]), jnp.float32),
    )(fm4, w_fc, b_fc[None, :])


def _rsqrt_nr(r2):
    i = lax.bitcast_convert_type(r2, jnp.int32)
    i = 1597463007 - lax.shift_right_arithmetic(i, 1)
    y = lax.bitcast_convert_type(i, jnp.float32)
    for _ in range(3):
        y = y * (1.5 - 0.5 * r2 * y * y)
    return y


def _make_sc_conv(vtab, vout, cdim, nch, nv, nn, has_theta, has_fsup,
                  has_center, relu):
    cper = cdim // nch
    vchunk = vout // nv
    vblk = min(vchunk, 128)
    nnpad = _NNPAD if has_theta else 16
    assert _B * nch * nv == 32
    assert cper % 16 == 0 and vchunk % vblk == 0

    mesh = plsc.VectorSubcoreMesh(core_axis_name="c", subcore_axis_name="s")

    ins = []
    if has_theta:
        ins += [jax.ShapeDtypeStruct((_B * vtab,), jnp.float32)] * 3
        ins.append(jax.ShapeDtypeStruct((nch, 3, cper), jnp.float32))
    ins.append(jax.ShapeDtypeStruct((_B, vout, nnpad), jnp.int32))
    if has_fsup:
        ins.append(jax.ShapeDtypeStruct((_B, nch, vtab, cper), jnp.float32))
    if has_center:
        ins.append(jax.ShapeDtypeStruct((_B, nch, vout, cper), jnp.float32))

    scratch = []
    if has_theta:
        scratch += [pltpu.VMEM((vtab,), jnp.float32)] * 3
        scratch += [pltpu.VMEM((3, cper), jnp.float32)]
    if has_fsup:
        scratch += [pltpu.VMEM((vtab, cper), jnp.float32)]
    scratch += [pltpu.VMEM((vblk, nnpad), jnp.int32)]
    if has_center:
        scratch += [pltpu.VMEM((vblk, cper), jnp.float32)]
    scratch += [pltpu.VMEM((vblk, cper), jnp.float32)]

    @functools.partial(
        pl.kernel,
        mesh=mesh,
        out_type=jax.ShapeDtypeStruct((_B, nch, vout, cper), jnp.float32),
        scratch_types=scratch,
        compiler_params=pltpu.CompilerParams(needs_layout_passes=False,
                                             use_tc_tiling_on_sc=False),
    )
    def conv(*refs):
        it = iter(refs)
        if has_theta:
            vx_hbm, vy_hbm, vz_hbm = next(it), next(it), next(it)
            sdn_hbm = next(it)
        nidx_hbm = next(it)
        fsup_hbm = next(it) if has_fsup else None
        cen_hbm = next(it) if has_center else None
        out_hbm = next(it)
        if has_theta:
            vx, vy, vz = next(it), next(it), next(it)
            sdn = next(it)
        fsup = next(it) if has_fsup else None
        nidx = next(it)
        cen = next(it) if has_center else None
        out = next(it)

        wid = lax.axis_index("s") * 2 + lax.axis_index("c")
        b = wid // (nch * nv)
        ci = (wid // nv) % nch
        vg = wid % nv

        if has_theta:
            pltpu.sync_copy(vx_hbm.at[pl.ds(b * vtab, vtab)], vx)
            pltpu.sync_copy(vy_hbm.at[pl.ds(b * vtab, vtab)], vy)
            pltpu.sync_copy(vz_hbm.at[pl.ds(b * vtab, vtab)], vz)
            pltpu.sync_copy(sdn_hbm.at[ci], sdn)
            for t in range(cper // 16):
                sl = pl.ds(t * 16, 16)
                sx, sy, sz = sdn[0, sl], sdn[1, sl], sdn[2, sl]
                inv = _rsqrt_nr(jnp.maximum(sx * sx + sy * sy + sz * sz,
                                            1e-24))
                sdn[0, sl] = sx * inv
                sdn[1, sl] = sy * inv
                sdn[2, sl] = sz * inv
        if has_fsup:
            pltpu.sync_copy(fsup_hbm.at[b, ci], fsup)

        for blk in range(vchunk // vblk):
            v0 = vg * vchunk + blk * vblk
            pltpu.sync_copy(nidx_hbm.at[b, pl.ds(v0, vblk)], nidx)
            if has_center:
                pltpu.sync_copy(cen_hbm.at[b, ci, pl.ds(v0, vblk)], cen)

            def per_vertex(i, _):
                ics = [nidx[i, pl.ds(j * 16, 16)]
                       for j in range(nnpad // 16)]
                dxv, dyv, dzv = [], [], []
                if has_theta:
                    g = jnp.full((16,), v0 + i, jnp.int32)
                    cxv = plsc.load_gather(vx, [g])
                    cyv = plsc.load_gather(vy, [g])
                    czv = plsc.load_gather(vz, [g])
                    for j in range(nnpad // 16):
                        dx = plsc.load_gather(vx, [ics[j]]) - cxv
                        dy = plsc.load_gather(vy, [ics[j]]) - cyv
                        dz = plsc.load_gather(vz, [ics[j]]) - czv
                        r2 = jnp.maximum(dx * dx + dy * dy + dz * dz, 1e-24)
                        inv = _rsqrt_nr(r2)
                        dxv.append(dx * inv)
                        dyv.append(dy * inv)
                        dzv.append(dz * inv)

                def per_chunk(t, _):
                    sl = pl.ds(t * 16, 16)
                    if has_theta:
                        sx, sy, sz = sdn[0, sl], sdn[1, sl], sdn[2, sl]
                    acc = jnp.full((16,), -_BIG, jnp.float32)
                    for n in range(nn):
                        j, l = divmod(n, 16)
                        if has_theta:
                            th = jnp.maximum(
                                dxv[j][l] * sx + dyv[j][l] * sy
                                + dzv[j][l] * sz, 0.0)
                        if has_fsup:
                            row = fsup[ics[j][l], sl]
                            val = th * row if has_theta else row
                        else:
                            val = th
                        acc = jnp.maximum(acc, val)
                    if has_center:
                        acc = acc + cen[i, sl]
                    if relu:
                        acc = jnp.maximum(acc, 0.0)
                    out[i, sl] = acc
                    return 0

                lax.fori_loop(0, cper // 16, per_chunk, 0)
                return 0

            lax.fori_loop(0, vblk, per_vertex, 0)
            pltpu.sync_copy(out, out_hbm.at[b, ci, pl.ds(v0, vblk)])

    return conv


_SC_SURF = _make_sc_conv(2048, 2048, 32, 1, 4, _NN, True, False, False, False)
_SC_CONV1 = _make_sc_conv(2048, 2048, 64, 2, 2, _NN, True, True, True, True)
_SC_POOL1 = _make_sc_conv(2048, 512, 64, 2, 2, 4, False, True, False, False)
_SC_CONV2 = _make_sc_conv(512, 512, 128, 1, 4, _NN, True, True, True, True)
_SC_CONV3 = _make_sc_conv(512, 512, 256, 2, 2, _NN, True, True, True, True)
_SC_POOL2 = _make_sc_conv(512, 128, 256, 2, 2, 4, False, True, False, False)
_SC_CONV4 = _make_sc_conv(128, 128, 1024, 4, 1, _NN, True, True, True, False)


def _arrange(x, nch):
    b, v, c = x.shape
    return jnp.transpose(x.reshape(b, v, nch, c // nch), (0, 2, 1, 3))


def _unarrange(x):
    b, nch, v, cper = x.shape
    return jnp.transpose(x, (0, 2, 1, 3)).reshape(b, v, nch * cper)


def _coords(v):
    return v[..., 0].reshape(-1), v[..., 1].reshape(-1), v[..., 2].reshape(-1)


def _sdn_arr(d, nch):
    return jnp.transpose(d.reshape(3, nch, d.shape[1] // nch), (1, 0, 2))


def kernel(vertices, directions_0, weights_1, bias_1, directions_1,
           weights_2, bias_2, directions_2, weights_3, bias_3, directions_3,
           weights_4, bias_4, directions_4, W_fc, b_fc):
    samp1 = jax.random.permutation(jax.random.key(42), 2048)[:512]
    samp2 = jax.random.permutation(jax.random.key(43), 512)[:128]

    v0 = vertices
    v1 = v0[:, samp1]
    v2 = v1[:, samp2]
    x0, y0, z0 = _coords(v0)
    x1, y1, z1 = _coords(v1)
    x2, y2, z2 = _coords(v2)

    nidx0 = _knn(v0)
    nidx1 = _knn(v1)
    nidx2 = _knn(v2)
    pidx1 = jnp.pad(nidx0[:, samp1, :4], ((0, 0), (0, 0), (0, 12)))
    pidx2 = jnp.pad(nidx1[:, samp2, :4], ((0, 0), (0, 0), (0, 12)))

    fm0 = _SC_SURF(x0, y0, z0, _sdn_arr(directions_0, 1),
                   nidx0).reshape(_B, 2048, 32)

    fo1 = _mm(fm0.reshape(-1, 32), weights_1, bias_1).reshape(_B, 2048, 128)
    fm1_arr = _SC_CONV1(x0, y0, z0, _sdn_arr(directions_1, 2), nidx0,
                        _arrange(fo1[..., 64:], 2), _arrange(fo1[..., :64], 2))
    fm1p = _unarrange(_SC_POOL1(pidx1, fm1_arr))

    fo2 = _mm(fm1p.reshape(-1, 64), weights_2, bias_2).reshape(_B, 512, 256)
    fm2 = _SC_CONV2(x1, y1, z1, _sdn_arr(directions_2, 1), nidx1,
                    _arrange(fo2[..., 128:], 1),
                    _arrange(fo2[..., :128], 1)).reshape(_B, 512, 128)

    fo3 = _mm(fm2.reshape(-1, 128), weights_3, bias_3).reshape(_B, 512, 512)
    fm3_arr = _SC_CONV3(x1, y1, z1, _sdn_arr(directions_3, 2), nidx1,
                        _arrange(fo3[..., 256:], 2),
                        _arrange(fo3[..., :256], 2))
    fm3p = _unarrange(_SC_POOL2(pidx2, fm3_arr))

    fo4 = _mm(fm3p.reshape(-1, 256), weights_4, bias_4).reshape(_B, 128, 2048)
    fm4 = _unarrange(_SC_CONV4(x2, y2, z2, _sdn_arr(directions_4, 4), nidx2,
                               _arrange(fo4[..., 1024:], 4),
                               _arrange(fo4[..., :1024], 4)))

    return _final(fm4, W_fc, b_fc)


# speedup vs baseline: 1.0845x; 1.0845x over previous
import functools
import math

import jax
import jax.numpy as jnp
from jax import lax
from jax.experimental import pallas as pl
from jax.experimental.pallas import tpu as pltpu
from jax.experimental.pallas import tpu_sc as plsc

_B = 8
_NN = 50
_NNPAD = 64
_BIG = 3.0e38


_MAXI = 2147483647


def _knn_body(xall_ref, xblk_ref, kmat_ref, aux_ref, *, rows, v, cw):
    xb = xblk_ref[0]
    qb = jnp.sum(xb * xb, axis=1)[:, None]
    rowg = pl.program_id(1) * rows + lax.broadcasted_iota(
        jnp.int32, (rows, cw), 0)
    nchunks = v // cw

    for c in range(nchunks):
        xa = xall_ref[0, pl.ds(c * cw, cw), :]
        inner = lax.dot_general(xb, xa, (((1,), (1,)), ((), ())),
                                preferred_element_type=jnp.float32)
        qa = jnp.sum(xa * xa, axis=1)[None, :]
        dist = jnp.maximum(qa + qb - 2.0 * inner, 0.0)
        bits = lax.bitcast_convert_type(dist, jnp.int32)
        colc = c * cw + lax.broadcasted_iota(jnp.int32, (rows, cw), 1)
        key = ((bits + 1024) & ~2047) | colc
        kmat_ref[0, :, pl.ds(c * cw, cw)] = jnp.where(colc == rowg, _MAXI,
                                                      key)

    col16 = lax.broadcasted_iota(jnp.int32, (rows, 16), 1)

    def step4(k, carry):
        res, m = carry
        acc = jnp.full((rows, 128), _MAXI, jnp.int32)
        for c in range(nchunks):
            kch = kmat_ref[0, :, pl.ds(c * cw, cw)]
            kch = jnp.where(kch > m, kch, _MAXI)
            for s in range(cw // 128):
                acc = jnp.minimum(acc, kch[:, s * 128:(s + 1) * 128])
        m = jnp.min(acc, axis=1, keepdims=True)
        return jnp.where(col16 == k, m & 2047, res), m

    res0 = jnp.zeros((rows, 16), jnp.int32)
    m0 = jnp.full((rows, 1), -1, jnp.int32)
    res, _ = lax.fori_loop(0, 4, step4, (res0, m0), unroll=1)

    def stepbit(t, mlo):
        cand = mlo | lax.shift_left(jnp.int32(1), 30 - t)
        acc = jnp.zeros((rows, 128), jnp.int32)
        for c in range(nchunks):
            kch = kmat_ref[0, :, pl.ds(c * cw, cw)]
            for s in range(cw // 128):
                acc = acc + (kch[:, s * 128:(s + 1) * 128]
                             <= cand).astype(jnp.int32)
        cnt = jnp.sum(acc, axis=1, keepdims=True)
        return jnp.where(cnt < _NN, cand, mlo)

    mlo = lax.fori_loop(0, 31, stepbit, jnp.zeros((rows, 1), jnp.int32),
                        unroll=1)
    aux_ref[0] = jnp.where(col16 == 4, mlo + 1, res)


def _knn(verts):
    b, v, _ = verts.shape
    rows = 64 if v > 512 else min(v, 128)
    cw = min(v, 512)
    vp = jnp.pad(verts, ((0, 0), (0, 0), (0, 125)))
    return pl.pallas_call(
        functools.partial(_knn_body, rows=rows, v=v, cw=cw),
        grid=(b, v // rows),
        in_specs=[
            pl.BlockSpec((1, v, 128), lambda i, j: (i, 0, 0)),
            pl.BlockSpec((1, rows, 128), lambda i, j: (i, j, 0)),
        ],
        out_specs=[
            pl.BlockSpec((1, rows, v), lambda i, j: (i, j, 0)),
            pl.BlockSpec((1, rows, 16), lambda i, j: (i, j, 0)),
        ],
        out_shape=[
            jax.ShapeDtypeStruct((b, v, v), jnp.int32),
            jax.ShapeDtypeStruct((b, v, 16), jnp.int32),
        ],
    )(vp, vp)


def _make_sc_compact(v, vblk):
    vchunk = v // 4
    assert vchunk % vblk == 0
    mesh = plsc.VectorSubcoreMesh(core_axis_name="c", subcore_axis_name="s")

    @functools.partial(
        pl.kernel,
        mesh=mesh,
        out_type=jax.ShapeDtypeStruct((_B, v, _NNPAD), jnp.int32),
        scratch_types=[
            pltpu.VMEM((vblk, v), jnp.int32),
            pltpu.VMEM((vblk, 16), jnp.int32),
            pltpu.VMEM((vblk, _NNPAD + 16), jnp.int32),
        ],
        compiler_params=pltpu.CompilerParams(needs_layout_passes=False,
                                             use_tc_tiling_on_sc=False),
    )
    def compact(kmat_hbm, aux_hbm, out_hbm, kbuf, abuf, obuf):
        wid = lax.axis_index("s") * 2 + lax.axis_index("c")
        b = wid // 4
        vg = wid % 4
        zeros16 = jnp.zeros((16,), jnp.int32)

        for blk in range(vchunk // vblk):
            v0 = vg * vchunk + blk * vblk
            pltpu.sync_copy(kmat_hbm.at[b, pl.ds(v0, vblk)], kbuf)
            pltpu.sync_copy(aux_hbm.at[b, pl.ds(v0, vblk)], abuf)

            def per_vertex(i, _):
                for t in range((_NNPAD + 16) // 16):
                    obuf[i, pl.ds(t * 16, 16)] = zeros16
                tau = abuf[i, pl.ds(0, 16)][4]

                def per_chunk(c, off):
                    kv = kbuf[i, pl.ds(c * 16, 16)]
                    msk = kv <= tau
                    plsc.store_compressed(obuf.at[i, pl.ds(off, 16)],
                                          kv & 2047, mask=msk)
                    return off + plsc.all_reduce_population_count(msk)[0]

                lax.fori_loop(0, v // 16, per_chunk, jnp.int32(0))
                return 0

            lax.fori_loop(0, vblk, per_vertex, 0)
            pltpu.sync_copy(obuf.at[:, pl.ds(0, _NNPAD)],
                            out_hbm.at[b, pl.ds(v0, vblk)])

    return compact


_SC_COMPACT = {2048: _make_sc_compact(2048, 32),
               512: _make_sc_compact(512, 64),
               128: _make_sc_compact(128, 32)}


def _mm_body(x_ref, w_ref, b_ref, o_ref):
    o_ref[...] = jnp.dot(x_ref[...], w_ref[...],
                         preferred_element_type=jnp.float32) + b_ref[...]


def _mm(x, w, b):
    m, kdim = x.shape
    n = w.shape[1]
    mblk = min(m, 2048)
    return pl.pallas_call(
        _mm_body,
        grid=(m // mblk,),
        in_specs=[
            pl.BlockSpec((mblk, kdim), lambda i: (i, 0)),
            pl.BlockSpec((kdim, n), lambda i: (0, 0)),
            pl.BlockSpec((1, n), lambda i: (0, 0)),
        ],
        out_specs=pl.BlockSpec((mblk, n), lambda i: (i, 0)),
        out_shape=jax.ShapeDtypeStruct((m, n), jnp.float32),
    )(x, w, b[None, :])


def _final_body(x_ref, w_ref, b_ref, o_ref):
    rows = []
    for i in range(_B):
        rows.append(jnp.max(x_ref[i], axis=0)[None, :])
    gmax = jnp.concatenate(rows, axis=0)
    o_ref[...] = jnp.dot(gmax, w_ref[...],
                         preferred_element_type=jnp.float32) + b_ref[...]


def _final(fm4, w_fc, b_fc):
    return pl.pallas_call(
        _final_body,
        out_shape=jax.ShapeDtypeStruct((_B, w_fc.shape[1]), jnp.float32),
    )(fm4, w_fc, b_fc[None, :])


def _rsqrt_nr(r2):
    i = lax.bitcast_convert_type(r2, jnp.int32)
    i = 1597463007 - lax.shift_right_arithmetic(i, 1)
    y = lax.bitcast_convert_type(i, jnp.float32)
    for _ in range(3):
        y = y * (1.5 - 0.5 * r2 * y * y)
    return y


def _make_sc_conv(vtab, vout, cdim, nch, nv, nn, has_theta, has_fsup,
                  has_center, relu):
    cper = cdim // nch
    vchunk = vout // nv
    vblk = min(vchunk, 128)
    nnpad = _NNPAD if has_theta else 16
    assert _B * nch * nv == 32
    assert cper % 16 == 0 and vchunk % vblk == 0

    mesh = plsc.VectorSubcoreMesh(core_axis_name="c", subcore_axis_name="s")

    ins = []
    if has_theta:
        ins += [jax.ShapeDtypeStruct((_B * vtab,), jnp.float32)] * 3
        ins.append(jax.ShapeDtypeStruct((nch, 3, cper), jnp.float32))
    ins.append(jax.ShapeDtypeStruct((_B, vout, nnpad), jnp.int32))
    if has_fsup:
        ins.append(jax.ShapeDtypeStruct((_B, nch, vtab, cper), jnp.float32))
    if has_center:
        ins.append(jax.ShapeDtypeStruct((_B, nch, vout, cper), jnp.float32))

    scratch = []
    if has_theta:
        scratch += [pltpu.VMEM((vtab,), jnp.float32)] * 3
        scratch += [pltpu.VMEM((3, cper), jnp.float32)]
    if has_fsup:
        scratch += [pltpu.VMEM((vtab, cper), jnp.float32)]
    scratch += [pltpu.VMEM((vblk, nnpad), jnp.int32)]
    if has_center:
        scratch += [pltpu.VMEM((vblk, cper), jnp.float32)]
    scratch += [pltpu.VMEM((vblk, cper), jnp.float32)]

    @functools.partial(
        pl.kernel,
        mesh=mesh,
        out_type=jax.ShapeDtypeStruct((_B, nch, vout, cper), jnp.float32),
        scratch_types=scratch,
        compiler_params=pltpu.CompilerParams(needs_layout_passes=False,
                                             use_tc_tiling_on_sc=False),
    )
    def conv(*refs):
        it = iter(refs)
        if has_theta:
            vx_hbm, vy_hbm, vz_hbm = next(it), next(it), next(it)
            sdn_hbm = next(it)
        nidx_hbm = next(it)
        fsup_hbm = next(it) if has_fsup else None
        cen_hbm = next(it) if has_center else None
        out_hbm = next(it)
        if has_theta:
            vx, vy, vz = next(it), next(it), next(it)
            sdn = next(it)
        fsup = next(it) if has_fsup else None
        nidx = next(it)
        cen = next(it) if has_center else None
        out = next(it)

        wid = lax.axis_index("s") * 2 + lax.axis_index("c")
        b = wid // (nch * nv)
        ci = (wid // nv) % nch
        vg = wid % nv

        if has_theta:
            pltpu.sync_copy(vx_hbm.at[pl.ds(b * vtab, vtab)], vx)
            pltpu.sync_copy(vy_hbm.at[pl.ds(b * vtab, vtab)], vy)
            pltpu.sync_copy(vz_hbm.at[pl.ds(b * vtab, vtab)], vz)
            pltpu.sync_copy(sdn_hbm.at[ci], sdn)
            for t in range(cper // 16):
                sl = pl.ds(t * 16, 16)
                sx, sy, sz = sdn[0, sl], sdn[1, sl], sdn[2, sl]
                inv = _rsqrt_nr(jnp.maximum(sx * sx + sy * sy + sz * sz,
                                            1e-24))
                sdn[0, sl] = sx * inv
                sdn[1, sl] = sy * inv
                sdn[2, sl] = sz * inv
        if has_fsup:
            pltpu.sync_copy(fsup_hbm.at[b, ci], fsup)

        for blk in range(vchunk // vblk):
            v0 = vg * vchunk + blk * vblk
            pltpu.sync_copy(nidx_hbm.at[b, pl.ds(v0, vblk)], nidx)
            if has_center:
                pltpu.sync_copy(cen_hbm.at[b, ci, pl.ds(v0, vblk)], cen)

            def per_vertex(i, _):
                ics = [nidx[i, pl.ds(j * 16, 16)]
                       for j in range(nnpad // 16)]
                dxv, dyv, dzv = [], [], []
                if has_theta:
                    g = jnp.full((16,), v0 + i, jnp.int32)
                    cxv = plsc.load_gather(vx, [g])
                    cyv = plsc.load_gather(vy, [g])
                    czv = plsc.load_gather(vz, [g])
                    for j in range(nnpad // 16):
                        dx = plsc.load_gather(vx, [ics[j]]) - cxv
                        dy = plsc.load_gather(vy, [ics[j]]) - cyv
                        dz = plsc.load_gather(vz, [ics[j]]) - czv
                        r2 = jnp.maximum(dx * dx + dy * dy + dz * dz, 1e-24)
                        inv = _rsqrt_nr(r2)
                        dxv.append(dx * inv)
                        dyv.append(dy * inv)
                        dzv.append(dz * inv)

                def per_chunk(t, _):
                    sl = pl.ds(t * 16, 16)
                    if has_theta:
                        sx, sy, sz = sdn[0, sl], sdn[1, sl], sdn[2, sl]
                    acc = jnp.full((16,), -_BIG, jnp.float32)
                    for n in range(nn):
                        j, l = divmod(n, 16)
                        if has_theta:
                            th = jnp.maximum(
                                dxv[j][l] * sx + dyv[j][l] * sy
                                + dzv[j][l] * sz, 0.0)
                        if has_fsup:
                            row = fsup[ics[j][l], sl]
                            val = th * row if has_theta else row
                        else:
                            val = th
                        acc = jnp.maximum(acc, val)
                    if has_center:
                        acc = acc + cen[i, sl]
                    if relu:
                        acc = jnp.maximum(acc, 0.0)
                    out[i, sl] = acc
                    return 0

                lax.fori_loop(0, cper // 16, per_chunk, 0)
                return 0

            lax.fori_loop(0, vblk, per_vertex, 0)
            pltpu.sync_copy(out, out_hbm.at[b, ci, pl.ds(v0, vblk)])

    return conv


_SC_SURF = _make_sc_conv(2048, 2048, 32, 1, 4, _NN, True, False, False, False)
_SC_CONV1 = _make_sc_conv(2048, 2048, 64, 2, 2, _NN, True, True, True, True)
_SC_POOL1 = _make_sc_conv(2048, 512, 64, 2, 2, 4, False, True, False, False)
_SC_CONV2 = _make_sc_conv(512, 512, 128, 1, 4, _NN, True, True, True, True)
_SC_CONV3 = _make_sc_conv(512, 512, 256, 2, 2, _NN, True, True, True, True)
_SC_POOL2 = _make_sc_conv(512, 128, 256, 2, 2, 4, False, True, False, False)
_SC_CONV4 = _make_sc_conv(128, 128, 1024, 4, 1, _NN, True, True, True, False)


def _arrange(x, nch):
    b, v, c = x.shape
    return jnp.transpose(x.reshape(b, v, nch, c // nch), (0, 2, 1, 3))


def _unarrange(x):
    b, nch, v, cper = x.shape
    return jnp.transpose(x, (0, 2, 1, 3)).reshape(b, v, nch * cper)


def _coords(v):
    return v[..., 0].reshape(-1), v[..., 1].reshape(-1), v[..., 2].reshape(-1)


def _sdn_arr(d, nch):
    return jnp.transpose(d.reshape(3, nch, d.shape[1] // nch), (1, 0, 2))


def kernel(vertices, directions_0, weights_1, bias_1, directions_1,
           weights_2, bias_2, directions_2, weights_3, bias_3, directions_3,
           weights_4, bias_4, directions_4, W_fc, b_fc):
    samp1 = jax.random.permutation(jax.random.key(42), 2048)[:512]
    samp2 = jax.random.permutation(jax.random.key(43), 512)[:128]

    v0 = vertices
    v1 = v0[:, samp1]
    v2 = v1[:, samp2]
    x0, y0, z0 = _coords(v0)
    x1, y1, z1 = _coords(v1)
    x2, y2, z2 = _coords(v2)

    kmat0, aux0 = _knn(v0)
    kmat1, aux1 = _knn(v1)
    kmat2, aux2 = _knn(v2)
    nidx0 = _SC_COMPACT[2048](kmat0, aux0)
    nidx1 = _SC_COMPACT[512](kmat1, aux1)
    nidx2 = _SC_COMPACT[128](kmat2, aux2)
    pidx1 = jnp.pad(aux0[:, samp1, :4], ((0, 0), (0, 0), (0, 12)))
    pidx2 = jnp.pad(aux1[:, samp2, :4], ((0, 0), (0, 0), (0, 12)))

    fm0 = _SC_SURF(x0, y0, z0, _sdn_arr(directions_0, 1),
                   nidx0).reshape(_B, 2048, 32)

    fo1 = _mm(fm0.reshape(-1, 32), weights_1, bias_1).reshape(_B, 2048, 128)
    fm1_arr = _SC_CONV1(x0, y0, z0, _sdn_arr(directions_1, 2), nidx0,
                        _arrange(fo1[..., 64:], 2), _arrange(fo1[..., :64], 2))
    fm1p = _unarrange(_SC_POOL1(pidx1, fm1_arr))

    fo2 = _mm(fm1p.reshape(-1, 64), weights_2, bias_2).reshape(_B, 512, 256)
    fm2 = _SC_CONV2(x1, y1, z1, _sdn_arr(directions_2, 1), nidx1,
                    _arrange(fo2[..., 128:], 1),
                    _arrange(fo2[..., :128], 1)).reshape(_B, 512, 128)

    fo3 = _mm(fm2.reshape(-1, 128), weights_3, bias_3).reshape(_B, 512, 512)
    fm3_arr = _SC_CONV3(x1, y1, z1, _sdn_arr(directions_3, 2), nidx1,
                        _arrange(fo3[..., 256:], 2),
                        _arrange(fo3[..., :256], 2))
    fm3p = _unarrange(_SC_POOL2(pidx2, fm3_arr))

    fo4 = _mm(fm3p.reshape(-1, 256), weights_4, bias_4).reshape(_B, 128, 2048)
    fm4 = _unarrange(_SC_CONV4(x2, y2, z2, _sdn_arr(directions_4, 4), nidx2,
                               _arrange(fo4[..., 1024:], 4),
                               _arrange(fo4[..., :1024], 4)))

    return _final(fm4, W_fc, b_fc)
